# Initial kernel scaffold; baseline (speedup 1.0000x reference)
import jax
import jax.numpy as jnp
from jax.experimental import pallas as pl


def kernel(x, pe):
    raise NotImplementedError("write your pallas kernel here")


# speedup vs baseline: 3.9489x; 3.9489x over previous
import jax
import jax.numpy as jnp
from jax.experimental import pallas as pl


def _tc_body(pe_ref, out_ref):
    blk = pe_ref[...]
    out_ref[...] = jnp.broadcast_to(blk[None], out_ref.shape)


def kernel(x, pe):
    batch, seq_len = x.shape
    d_model = pe.shape[1]
    block_s = 512
    grid = (seq_len // block_s,)
    return pl.pallas_call(
        _tc_body,
        grid=grid,
        in_specs=[pl.BlockSpec((block_s, d_model), lambda i: (i, 0))],
        out_specs=pl.BlockSpec((batch, block_s, d_model), lambda i: (0, i, 0)),
        out_shape=jax.ShapeDtypeStruct((batch, seq_len, d_model), jnp.float32),
    )(pe)
